# baseline (speedup 1.0000x reference)
import functools

import jax
import jax.numpy as jnp
from jax import lax
from jax.experimental import pallas as pl
from jax.experimental.pallas import tpu as pltpu
from jax.experimental.pallas import tpu_sc as plsc

_N = 10000
_E = 320000
_D = 128
_K = 1024
_NP = 10240
_RPW = _NP // 32
_SCH = 512
_NCH = _E // _SCH
_GCH = 128
_NGC = _E // _GCH
_NW = 32
_SEL = 512


def _lane0():
  return lax.iota(jnp.int32, 16) == 0


def _prefix16(v):
  lanes = lax.iota(jnp.int32, 16)
  dnums = lax.GatherDimensionNumbers(
      offset_dims=(), collapsed_slice_dims=(0,), start_index_map=(0,))
  for s in (1, 2, 4, 8):
    idx = jnp.maximum(lanes - s, 0)
    g = lax.gather(v, idx[:, None], dnums, (1,),
                   mode=lax.GatherScatterMode.PROMISE_IN_BOUNDS)
    v = v + jnp.where(lanes >= s, g, 0)
  return v


def _worker_id():
  ci = lax.axis_index("c")
  si = lax.axis_index("s")
  return si * 2 + ci, ci, si


def _sc_fold_body(h_hbm, eac_hbm, dstc_hbm, srcc_hbm, zr_hbm, zd_hbm,
                  acc_out, deg_out,
                  dstb_v, srcb_v, seld_v, sels_v, selp_v, cntv_v,
                  rowa_v, rowb_v, accl_v, degl_v, sem1, sem2):
  w, ci, si = _worker_id()
  lo = w * _RPW
  pltpu.sync_copy(zr_hbm, accl_v)
  pltpu.sync_copy(zd_hbm, degl_v)
  z16 = jnp.zeros((16,), jnp.int32)

  def zinit(k, carry):
    sels_v[pl.ds(k * 16, 16)] = z16
    selp_v[pl.ds(k * 16, 16)] = z16
    return carry

  lax.fori_loop(0, _SEL // 16, zinit, 0)

  def fold_first(nfold):
    cpa = pltpu.async_copy(h_hbm.at[sels_v.at[pl.ds(0, 128)]], rowa_v, sem1)
    cpb = pltpu.async_copy(eac_hbm.at[selp_v.at[pl.ds(0, 128)]], rowb_v, sem2)
    cpa.wait()
    cpb.wait()

    def fold_one(i, carry):
      @pl.when(i < nfold)
      def _():
        dloc = seld_v[pl.ds(i, 16)][0]
        plsc.addupdate_scatter(degl_v, [jnp.full((16,), dloc, jnp.int32)],
                               jnp.ones((16,), jnp.float32), mask=_lane0())
        for jj in range(8):
          sl = pl.ds(jj * 16, 16)
          plsc.addupdate(accl_v.at[dloc, sl], rowa_v[i, sl] + rowb_v[i, sl])
      return carry

    lax.fori_loop(0, 128, fold_one, 0)

  def shift_sel(k, carry):
    sl_from = pl.ds(128 + k * 16, 16)
    sl_to = pl.ds(k * 16, 16)
    seld_v[sl_to] = seld_v[sl_from]
    sels_v[sl_to] = sels_v[sl_from]
    selp_v[sl_to] = selp_v[sl_from]
    return carry

  def chunk_body(c, cnt):
    base = c * _SCH
    pltpu.sync_copy(dstc_hbm.at[pl.ds(base, _SCH)], dstb_v)
    pltpu.sync_copy(srcc_hbm.at[pl.ds(base, _SCH)], srcb_v)

    def subvec(j, cnt):
      sl = pl.ds(j * 16, 16)
      d = dstb_v[sl]
      m = (d >= lo) & (d < lo + _RPW)
      pos = lax.iota(jnp.int32, 16) + (base + j * 16)
      csum = _prefix16(m.astype(jnp.int32))
      slot = cnt + csum - 1
      plsc.store_scatter(seld_v, [slot], d - lo, mask=m)
      plsc.store_scatter(sels_v, [slot], srcb_v[sl], mask=m)
      plsc.store_scatter(selp_v, [slot], pos, mask=m)
      cntv_v[pl.ds(0, 16)] = csum
      cnt = cnt + cntv_v[pl.ds(0, 16)][15]
      isfull = cnt >= 128

      @pl.when(isfull)
      def _():
        fold_first(jnp.int32(128))
        shifter = lax.fori_loop(0, 8, shift_sel, 0)

      return jnp.where(isfull, cnt - 128, cnt)

    return lax.fori_loop(0, _SCH // 16, subvec, cnt)

  cnt = lax.fori_loop(0, 2 * _NCH, chunk_body, jnp.int32(0))
  fold_first(cnt)

  pltpu.sync_copy(accl_v, acc_out.at[pl.ds(lo, _RPW), :])
  pltpu.sync_copy(degl_v, deg_out.at[pl.ds(lo, _RPW)])


def _sc_gather4_body(h1_hbm, h2_hbm, hid_hbm, tid_hbm,
                     gh1_out, gh2_out, gt1_out, gt2_out,
                     idxh_v, idxt_v, b1_v, b2_v, b3_v, b4_v,
                     sem1, sem2, sem3, sem4):
  w, ci, si = _worker_id()

  def body(j, carry):
    c = w + _NW * j
    base = c * _GCH
    pltpu.sync_copy(hid_hbm.at[pl.ds(base, _GCH)], idxh_v)
    pltpu.sync_copy(tid_hbm.at[pl.ds(base, _GCH)], idxt_v)
    cp1 = pltpu.async_copy(h1_hbm.at[idxh_v], b1_v, sem1)
    cp2 = pltpu.async_copy(h2_hbm.at[idxh_v], b2_v, sem2)
    cp3 = pltpu.async_copy(h1_hbm.at[idxt_v], b3_v, sem3)
    cp4 = pltpu.async_copy(h2_hbm.at[idxt_v], b4_v, sem4)
    cp1.wait()
    pltpu.sync_copy(b1_v, gh1_out.at[pl.ds(base, _GCH), :])
    cp2.wait()
    pltpu.sync_copy(b2_v, gh2_out.at[pl.ds(base, _GCH), :])
    cp3.wait()
    pltpu.sync_copy(b3_v, gt1_out.at[pl.ds(base, _GCH), :])
    cp4.wait()
    pltpu.sync_copy(b4_v, gt2_out.at[pl.ds(base, _GCH), :])
    return carry

  nc = 78 + jnp.where(w < 4, 1, 0).astype(jnp.int32)
  lax.fori_loop(0, nc, body, 0)


@functools.cache
def _sc_kernels():
  mesh = plsc.VectorSubcoreMesh(core_axis_name="c", subcore_axis_name="s",
                                num_cores=2, num_subcores=16)
  params = pltpu.CompilerParams(needs_layout_passes=False)
  fold = functools.partial(
      pl.kernel,
      mesh=mesh,
      compiler_params=params,
      out_type=(
          jax.ShapeDtypeStruct((_NP, _D), jnp.float32),
          jax.ShapeDtypeStruct((_NP,), jnp.float32),
      ),
      scratch_types=[
          pltpu.VMEM((_SCH,), jnp.int32),
          pltpu.VMEM((_SCH,), jnp.int32),
          pltpu.VMEM((_SEL,), jnp.int32),
          pltpu.VMEM((_SEL,), jnp.int32),
          pltpu.VMEM((_SEL,), jnp.int32),
          pltpu.VMEM((16,), jnp.int32),
          pltpu.VMEM((128, _D), jnp.float32),
          pltpu.VMEM((128, _D), jnp.float32),
          pltpu.VMEM((_RPW, _D), jnp.float32),
          pltpu.VMEM((_RPW,), jnp.float32),
          pltpu.SemaphoreType.DMA,
          pltpu.SemaphoreType.DMA,
      ],
  )(_sc_fold_body)
  gather4 = functools.partial(
      pl.kernel,
      mesh=mesh,
      compiler_params=params,
      out_type=tuple(
          jax.ShapeDtypeStruct((_E, _D), jnp.float32) for _ in range(4)),
      scratch_types=[
          pltpu.VMEM((_GCH,), jnp.int32),
          pltpu.VMEM((_GCH,), jnp.int32),
          pltpu.VMEM((_GCH, _D), jnp.float32),
          pltpu.VMEM((_GCH, _D), jnp.float32),
          pltpu.VMEM((_GCH, _D), jnp.float32),
          pltpu.VMEM((_GCH, _D), jnp.float32),
          pltpu.SemaphoreType.DMA,
          pltpu.SemaphoreType.DMA,
          pltpu.SemaphoreType.DMA,
          pltpu.SemaphoreType.DMA,
      ],
  )(_sc_gather4_body)
  return fold, gather4


def _mlp2_body(x_ref, w1_ref, b1_ref, w2_ref, b2_ref, o_ref):
  h = jnp.maximum(
      jnp.dot(x_ref[...], w1_ref[...], preferred_element_type=jnp.float32)
      + b1_ref[...], 0.0)
  o_ref[...] = (
      jnp.dot(h, w2_ref[...], preferred_element_type=jnp.float32)
      + b2_ref[...])


def _edge_mlp(ea, w1, b1, w2, b2, rb):
  n = ea.shape[0]
  full = lambda i: (0, 0)
  return pl.pallas_call(
      _mlp2_body,
      grid=(n // rb,),
      in_specs=[
          pl.BlockSpec((rb, _D), lambda i: (i, 0)),
          pl.BlockSpec((_D, _D), full),
          pl.BlockSpec((1, _D), full),
          pl.BlockSpec((_D, _D), full),
          pl.BlockSpec((1, _D), full),
      ],
      out_specs=pl.BlockSpec((rb, _D), lambda i: (i, 0)),
      out_shape=jax.ShapeDtypeStruct((n, _D), jnp.float32),
  )(ea, w1, b1, w2, b2)


def _layer_body(h_ref, a_ref, d_ref, ws, wn, b, o_ref):
  deg = jnp.maximum(d_ref[...], 1.0)
  agg = a_ref[...] / deg
  o_ref[...] = jnp.maximum(
      jnp.dot(h_ref[...], ws[...], preferred_element_type=jnp.float32)
      + jnp.dot(agg, wn[...], preferred_element_type=jnp.float32)
      + b[...], 0.0)


def _layer_combine(h, acc, deg, ws, wn, b, rb):
  full = lambda i: (0, 0)
  blk = pl.BlockSpec((rb, _D), lambda i: (i, 0))
  return pl.pallas_call(
      _layer_body,
      grid=(_NP // rb,),
      in_specs=[blk, blk, pl.BlockSpec((rb, 1), lambda i: (i, 0)),
                pl.BlockSpec((_D, _D), full),
                pl.BlockSpec((_D, _D), full),
                pl.BlockSpec((1, _D), full)],
      out_specs=blk,
      out_shape=jax.ShapeDtypeStruct((_NP, _D), jnp.float32),
  )(h, acc, deg, ws, wn, b)


def _score_body(bq_ref, gh1_ref, gh2_ref, ea_ref, gt1_ref, gt2_ref, gn_ref,
                wp1, bp1, wp2, bp2, logit_ref, y_ref):
  ht = jnp.concatenate(
      [bq_ref[...], gh1_ref[...], gh2_ref[...], ea_ref[...],
       gt1_ref[...], gt2_ref[...]], axis=1)
  hid = jnp.maximum(
      jnp.dot(ht, wp1[...], preferred_element_type=jnp.float32)
      + bp1[...], 0.0)
  lg = jnp.dot(hid, wp2[...], preferred_element_type=jnp.float32) + bp2[...]
  logit_ref[...] = lg
  y_ref[...] = lg + gn_ref[...]


def _edge_score(bq, gh1, gh2, ea, gt1, gt2, gn, wp1, bp1, wp2, bp2, rb):
  full = lambda i: (0, 0)
  blk = pl.BlockSpec((rb, _D), lambda i: (i, 0))
  col = pl.BlockSpec((rb, 1), lambda i: (i, 0))
  out = jax.ShapeDtypeStruct((_E, 1), jnp.float32)
  return pl.pallas_call(
      _score_body,
      grid=(_E // rb,),
      in_specs=[blk, blk, blk, blk, blk, blk, col,
                pl.BlockSpec((6 * _D, _D), full),
                pl.BlockSpec((1, _D), full),
                pl.BlockSpec((_D, 1), full),
                pl.BlockSpec((1, 1), full)],
      out_specs=(col, col),
      out_shape=(out, out),
  )(bq, gh1, gh2, ea, gt1, gt2, gn, wp1, bp1, wp2, bp2)


_EPAD = 2560 * 128


def _topk_body(y_ref, o_ref):
  zi = lax.bitcast_convert_type(y_ref[...], jnp.int32)
  z = zi ^ ((zi >> 31) & jnp.int32(0x7FFFFFFF))

  def bis1(i, c):
    lo, hi = c
    half = (lo >> 1) + (hi >> 1)
    mid0 = half + (lo & hi & 1) + ((lo ^ hi) & 1)
    mid = jnp.where(lo < hi, mid0, lo)
    cnt = jnp.sum((z >= mid).astype(jnp.int32))
    pred = cnt >= _K
    return (jnp.where(pred, mid, lo), jnp.where(pred, hi, mid - 1))

  lo, _ = lax.fori_loop(0, 33, bis1, (jnp.int32(-(2 ** 31)),
                                      jnp.int32(2 ** 31 - 1)))
  t = lo
  c_gt = jnp.sum((z > t).astype(jnp.int32))
  r = _K - c_gt
  eq = z == t
  rows = lax.broadcasted_iota(jnp.int32, z.shape, 0)
  cols = lax.broadcasted_iota(jnp.int32, z.shape, 1)
  flat = rows * 128 + cols

  def bis2(i, c):
    lo, hi = c
    mid = jnp.where(lo < hi, (lo + hi) >> 1, lo)
    cnt = jnp.sum((eq & (flat <= mid)).astype(jnp.int32))
    ok = cnt >= r
    return (jnp.where(ok, lo, mid + 1), jnp.where(ok, mid, hi))

  m, _ = lax.fori_loop(0, 20, bis2, (jnp.int32(0), jnp.int32(_EPAD - 1)))
  o_ref[...] = ((z > t) | (eq & (flat <= m))).astype(jnp.float32)


def _topk_mask(ypad):
  return pl.pallas_call(
      _topk_body,
      out_shape=jax.ShapeDtypeStruct((_EPAD // 128, 128), jnp.float32),
  )(ypad)


def kernel(x, edge_index, edge_attr, batch_q_embds, W_pr1, b_pr1, W_pr2,
           b_pr2, W_s1, b_s1, W_n1, W_s2, b_s2, W_n2, W_p1, b_p1, W_p2, b_p2):
  h_id = edge_index[0]
  t_id = edge_index[1]
  r1 = lambda v: v.reshape(1, -1)

  u = jax.random.uniform(jax.random.key(42), (_E,), jnp.float32,
                         1e-10, 1.0 - 1e-10)
  gn = (-jnp.log(-jnp.log(u))).reshape(_E, 1)

  _sc_fold, _sc_gather4 = _sc_kernels()

  ea_rev = _edge_mlp(edge_attr, W_pr1, r1(b_pr1), W_pr2, r1(b_pr2), 2560)

  zr = jnp.zeros((_RPW, _D), jnp.float32)
  zd = jnp.zeros((_RPW,), jnp.float32)
  xp = jnp.zeros((_NP, _D), jnp.float32).at[:_N].set(x)
  eacat = jnp.concatenate([edge_attr, ea_rev], axis=0)
  dstcat = jnp.concatenate([t_id, h_id])
  srccat = jnp.concatenate([h_id, t_id])

  acc1, deg = _sc_fold(xp, eacat, dstcat, srccat, zr, zd)
  degc = deg.reshape(_NP, 1)
  h1 = _layer_combine(xp, acc1, degc, W_s1, W_n1, r1(b_s1), 1280)

  acc2, _ = _sc_fold(h1, eacat, dstcat, srccat, zr, zd)
  h2 = _layer_combine(h1, acc2, degc, W_s2, W_n2, r1(b_s2), 1280)

  gh1, gh2, gt1, gt2 = _sc_gather4(h1, h2, h_id, t_id)

  logits2d, y2d = _edge_score(
      batch_q_embds, gh1, gh2, edge_attr, gt1, gt2, gn,
      W_p1, r1(b_p1), W_p2, b_p2.reshape(1, 1), 2560)

  ypad = jnp.concatenate(
      [y2d[:, 0], jnp.full((_EPAD - _E,), -jnp.inf, jnp.float32)]
  ).reshape(_EPAD // 128, 128)
  mask2d = _topk_mask(ypad)

  logits = logits2d[:, 0]
  mask = mask2d.reshape(-1)[:_E]
  return (logits, mask)


# speedup vs baseline: 1.1482x; 1.1482x over previous
import functools

import jax
import jax.numpy as jnp
from jax import lax
from jax.experimental import pallas as pl
from jax.experimental.pallas import tpu as pltpu
from jax.experimental.pallas import tpu_sc as plsc

_N = 10000
_E = 320000
_D = 128
_K = 1024
_NP = 10240
_RPW = _NP // 32
_SCH = 2048
_NCH = _E // _SCH
_GCH = 128
_NGC = _E // _GCH
_NW = 32
_SEL = 512


def _lane0():
  return lax.iota(jnp.int32, 16) == 0


def _prefix16(v):
  lanes = lax.iota(jnp.int32, 16)
  dnums = lax.GatherDimensionNumbers(
      offset_dims=(), collapsed_slice_dims=(0,), start_index_map=(0,))
  for s in (1, 2, 4, 8):
    idx = jnp.maximum(lanes - s, 0)
    g = lax.gather(v, idx[:, None], dnums, (1,),
                   mode=lax.GatherScatterMode.PROMISE_IN_BOUNDS)
    v = v + jnp.where(lanes >= s, g, 0)
  return v


def _worker_id():
  ci = lax.axis_index("c")
  si = lax.axis_index("s")
  return si * 2 + ci, ci, si


def _sc_fold_body(h_hbm, eac_hbm, dstc_hbm, srcc_hbm, zr_hbm, zd_hbm,
                  acc_out, deg_out,
                  dstb_v, srcb_v, seld_v, sels_v, selp_v, cntv_v,
                  rowa_v, rowb_v, accl_v, degl_v, sem1, sem2):
  w, ci, si = _worker_id()
  lo = w * _RPW
  pltpu.sync_copy(zr_hbm, accl_v)
  pltpu.sync_copy(zd_hbm, degl_v)
  z16 = jnp.zeros((16,), jnp.int32)

  def zinit(k, carry):
    sels_v[pl.ds(k * 16, 16)] = z16
    selp_v[pl.ds(k * 16, 16)] = z16
    return carry

  lax.fori_loop(0, _SEL // 16, zinit, 0)

  def fold_first(nfold):
    cpa = pltpu.async_copy(h_hbm.at[sels_v.at[pl.ds(0, 128)]], rowa_v, sem1)
    cpb = pltpu.async_copy(eac_hbm.at[selp_v.at[pl.ds(0, 128)]], rowb_v, sem2)
    cpa.wait()
    cpb.wait()

    def fold_one(i, carry):
      @pl.when(i < nfold)
      def _():
        dloc = seld_v[pl.ds(i, 16)][0]
        plsc.addupdate_scatter(degl_v, [jnp.full((16,), dloc, jnp.int32)],
                               jnp.ones((16,), jnp.float32), mask=_lane0())
        for jj in range(8):
          sl = pl.ds(jj * 16, 16)
          plsc.addupdate(accl_v.at[dloc, sl], rowa_v[i, sl] + rowb_v[i, sl])
      return carry

    lax.fori_loop(0, 128, fold_one, 0)

  def shift_sel(k, carry):
    sl_from = pl.ds(128 + k * 16, 16)
    sl_to = pl.ds(k * 16, 16)
    seld_v[sl_to] = seld_v[sl_from]
    sels_v[sl_to] = sels_v[sl_from]
    selp_v[sl_to] = selp_v[sl_from]
    return carry

  def chunk_body(c, cnt):
    base = c * _SCH
    pltpu.sync_copy(dstc_hbm.at[pl.ds(base, _SCH)], dstb_v)
    pltpu.sync_copy(srcc_hbm.at[pl.ds(base, _SCH)], srcb_v)

    def subvec(j, cnt):
      sl = pl.ds(j * 16, 16)
      d = dstb_v[sl]
      m = (d >= lo) & (d < lo + _RPW)
      cadd = plsc.all_reduce_population_count(m)[0]

      @pl.when(cadd > 0)
      def _():
        pos = lax.iota(jnp.int32, 16) + (base + j * 16)
        csum = _prefix16(m.astype(jnp.int32))
        slot = cnt + csum - 1
        plsc.store_scatter(seld_v, [slot], d - lo, mask=m)
        plsc.store_scatter(sels_v, [slot], srcb_v[sl], mask=m)
        plsc.store_scatter(selp_v, [slot], pos, mask=m)

      cnt = cnt + cadd
      isfull = cnt >= 128

      @pl.when(isfull)
      def _():
        fold_first(jnp.int32(128))
        shifter = lax.fori_loop(0, 8, shift_sel, 0)

      return jnp.where(isfull, cnt - 128, cnt)

    return lax.fori_loop(0, _SCH // 16, subvec, cnt)

  cnt = lax.fori_loop(0, 2 * _NCH, chunk_body, jnp.int32(0))
  fold_first(cnt)

  pltpu.sync_copy(accl_v, acc_out.at[pl.ds(lo, _RPW), :])
  pltpu.sync_copy(degl_v, deg_out.at[pl.ds(lo, _RPW)])


def _sc_gather4_body(h1_hbm, h2_hbm, hid_hbm, tid_hbm,
                     gh1_out, gh2_out, gt1_out, gt2_out,
                     idxh_v, idxt_v, b1_v, b2_v, b3_v, b4_v,
                     sem1, sem2, sem3, sem4):
  w, ci, si = _worker_id()

  def body(j, carry):
    c = w + _NW * j
    base = c * _GCH
    pltpu.sync_copy(hid_hbm.at[pl.ds(base, _GCH)], idxh_v)
    pltpu.sync_copy(tid_hbm.at[pl.ds(base, _GCH)], idxt_v)
    cp1 = pltpu.async_copy(h1_hbm.at[idxh_v], b1_v, sem1)
    cp2 = pltpu.async_copy(h2_hbm.at[idxh_v], b2_v, sem2)
    cp3 = pltpu.async_copy(h1_hbm.at[idxt_v], b3_v, sem3)
    cp4 = pltpu.async_copy(h2_hbm.at[idxt_v], b4_v, sem4)
    cp1.wait()
    pltpu.sync_copy(b1_v, gh1_out.at[pl.ds(base, _GCH), :])
    cp2.wait()
    pltpu.sync_copy(b2_v, gh2_out.at[pl.ds(base, _GCH), :])
    cp3.wait()
    pltpu.sync_copy(b3_v, gt1_out.at[pl.ds(base, _GCH), :])
    cp4.wait()
    pltpu.sync_copy(b4_v, gt2_out.at[pl.ds(base, _GCH), :])
    return carry

  nc = 78 + jnp.where(w < 4, 1, 0).astype(jnp.int32)
  lax.fori_loop(0, nc, body, 0)


@functools.cache
def _sc_kernels():
  mesh = plsc.VectorSubcoreMesh(core_axis_name="c", subcore_axis_name="s",
                                num_cores=2, num_subcores=16)
  params = pltpu.CompilerParams(needs_layout_passes=False)
  fold = functools.partial(
      pl.kernel,
      mesh=mesh,
      compiler_params=params,
      out_type=(
          jax.ShapeDtypeStruct((_NP, _D), jnp.float32),
          jax.ShapeDtypeStruct((_NP,), jnp.float32),
      ),
      scratch_types=[
          pltpu.VMEM((_SCH,), jnp.int32),
          pltpu.VMEM((_SCH,), jnp.int32),
          pltpu.VMEM((_SEL,), jnp.int32),
          pltpu.VMEM((_SEL,), jnp.int32),
          pltpu.VMEM((_SEL,), jnp.int32),
          pltpu.VMEM((16,), jnp.int32),
          pltpu.VMEM((128, _D), jnp.float32),
          pltpu.VMEM((128, _D), jnp.float32),
          pltpu.VMEM((_RPW, _D), jnp.float32),
          pltpu.VMEM((_RPW,), jnp.float32),
          pltpu.SemaphoreType.DMA,
          pltpu.SemaphoreType.DMA,
      ],
  )(_sc_fold_body)
  gather4 = functools.partial(
      pl.kernel,
      mesh=mesh,
      compiler_params=params,
      out_type=tuple(
          jax.ShapeDtypeStruct((_E, _D), jnp.float32) for _ in range(4)),
      scratch_types=[
          pltpu.VMEM((_GCH,), jnp.int32),
          pltpu.VMEM((_GCH,), jnp.int32),
          pltpu.VMEM((_GCH, _D), jnp.float32),
          pltpu.VMEM((_GCH, _D), jnp.float32),
          pltpu.VMEM((_GCH, _D), jnp.float32),
          pltpu.VMEM((_GCH, _D), jnp.float32),
          pltpu.SemaphoreType.DMA,
          pltpu.SemaphoreType.DMA,
          pltpu.SemaphoreType.DMA,
          pltpu.SemaphoreType.DMA,
      ],
  )(_sc_gather4_body)
  return fold, gather4


def _mlp2_body(x_ref, w1_ref, b1_ref, w2_ref, b2_ref, o_ref):
  h = jnp.maximum(
      jnp.dot(x_ref[...], w1_ref[...], preferred_element_type=jnp.float32)
      + b1_ref[...], 0.0)
  o_ref[...] = (
      jnp.dot(h, w2_ref[...], preferred_element_type=jnp.float32)
      + b2_ref[...])


def _edge_mlp(ea, w1, b1, w2, b2, rb):
  n = ea.shape[0]
  full = lambda i: (0, 0)
  return pl.pallas_call(
      _mlp2_body,
      grid=(n // rb,),
      in_specs=[
          pl.BlockSpec((rb, _D), lambda i: (i, 0)),
          pl.BlockSpec((_D, _D), full),
          pl.BlockSpec((1, _D), full),
          pl.BlockSpec((_D, _D), full),
          pl.BlockSpec((1, _D), full),
      ],
      out_specs=pl.BlockSpec((rb, _D), lambda i: (i, 0)),
      out_shape=jax.ShapeDtypeStruct((n, _D), jnp.float32),
  )(ea, w1, b1, w2, b2)


def _layer_body(h_ref, a_ref, d_ref, ws, wn, b, o_ref):
  deg = jnp.maximum(d_ref[...], 1.0)
  agg = a_ref[...] / deg
  o_ref[...] = jnp.maximum(
      jnp.dot(h_ref[...], ws[...], preferred_element_type=jnp.float32)
      + jnp.dot(agg, wn[...], preferred_element_type=jnp.float32)
      + b[...], 0.0)


def _layer_combine(h, acc, deg, ws, wn, b, rb):
  full = lambda i: (0, 0)
  blk = pl.BlockSpec((rb, _D), lambda i: (i, 0))
  return pl.pallas_call(
      _layer_body,
      grid=(_NP // rb,),
      in_specs=[blk, blk, pl.BlockSpec((rb, 1), lambda i: (i, 0)),
                pl.BlockSpec((_D, _D), full),
                pl.BlockSpec((_D, _D), full),
                pl.BlockSpec((1, _D), full)],
      out_specs=blk,
      out_shape=jax.ShapeDtypeStruct((_NP, _D), jnp.float32),
  )(h, acc, deg, ws, wn, b)


def _score_body(bq_ref, gh1_ref, gh2_ref, ea_ref, gt1_ref, gt2_ref, gn_ref,
                wp1, bp1, wp2, bp2, logit_ref, y_ref):
  ht = jnp.concatenate(
      [bq_ref[...], gh1_ref[...], gh2_ref[...], ea_ref[...],
       gt1_ref[...], gt2_ref[...]], axis=1)
  hid = jnp.maximum(
      jnp.dot(ht, wp1[...], preferred_element_type=jnp.float32)
      + bp1[...], 0.0)
  lg = jnp.dot(hid, wp2[...], preferred_element_type=jnp.float32) + bp2[...]
  logit_ref[...] = lg
  y_ref[...] = lg + gn_ref[...]


def _edge_score(bq, gh1, gh2, ea, gt1, gt2, gn, wp1, bp1, wp2, bp2, rb):
  full = lambda i: (0, 0)
  blk = pl.BlockSpec((rb, _D), lambda i: (i, 0))
  col = pl.BlockSpec((rb, 1), lambda i: (i, 0))
  out = jax.ShapeDtypeStruct((_E, 1), jnp.float32)
  return pl.pallas_call(
      _score_body,
      grid=(_E // rb,),
      in_specs=[blk, blk, blk, blk, blk, blk, col,
                pl.BlockSpec((6 * _D, _D), full),
                pl.BlockSpec((1, _D), full),
                pl.BlockSpec((_D, 1), full),
                pl.BlockSpec((1, 1), full)],
      out_specs=(col, col),
      out_shape=(out, out),
  )(bq, gh1, gh2, ea, gt1, gt2, gn, wp1, bp1, wp2, bp2)


_EPAD = 2560 * 128


def _topk_body(y_ref, o_ref):
  zi = lax.bitcast_convert_type(y_ref[...], jnp.int32)
  z = zi ^ ((zi >> 31) & jnp.int32(0x7FFFFFFF))

  def bis1(i, c):
    lo, hi = c
    half = (lo >> 1) + (hi >> 1)
    mid0 = half + (lo & hi & 1) + ((lo ^ hi) & 1)
    mid = jnp.where(lo < hi, mid0, lo)
    cnt = jnp.sum((z >= mid).astype(jnp.int32))
    pred = cnt >= _K
    return (jnp.where(pred, mid, lo), jnp.where(pred, hi, mid - 1))

  lo, _ = lax.fori_loop(0, 33, bis1, (jnp.int32(-(2 ** 31)),
                                      jnp.int32(2 ** 31 - 1)))
  t = lo
  c_gt = jnp.sum((z > t).astype(jnp.int32))
  r = _K - c_gt
  eq = z == t
  rows = lax.broadcasted_iota(jnp.int32, z.shape, 0)
  cols = lax.broadcasted_iota(jnp.int32, z.shape, 1)
  flat = rows * 128 + cols

  def bis2(i, c):
    lo, hi = c
    mid = jnp.where(lo < hi, (lo + hi) >> 1, lo)
    cnt = jnp.sum((eq & (flat <= mid)).astype(jnp.int32))
    ok = cnt >= r
    return (jnp.where(ok, lo, mid + 1), jnp.where(ok, mid, hi))

  m, _ = lax.fori_loop(0, 20, bis2, (jnp.int32(0), jnp.int32(_EPAD - 1)))
  o_ref[...] = ((z > t) | (eq & (flat <= m))).astype(jnp.float32)


def _topk_mask(ypad):
  return pl.pallas_call(
      _topk_body,
      out_shape=jax.ShapeDtypeStruct((_EPAD // 128, 128), jnp.float32),
  )(ypad)


def kernel(x, edge_index, edge_attr, batch_q_embds, W_pr1, b_pr1, W_pr2,
           b_pr2, W_s1, b_s1, W_n1, W_s2, b_s2, W_n2, W_p1, b_p1, W_p2, b_p2):
  h_id = edge_index[0]
  t_id = edge_index[1]
  r1 = lambda v: v.reshape(1, -1)

  u = jax.random.uniform(jax.random.key(42), (_E,), jnp.float32,
                         1e-10, 1.0 - 1e-10)
  gn = (-jnp.log(-jnp.log(u))).reshape(_E, 1)

  _sc_fold, _sc_gather4 = _sc_kernels()

  ea_rev = _edge_mlp(edge_attr, W_pr1, r1(b_pr1), W_pr2, r1(b_pr2), 2560)

  zr = jnp.zeros((_RPW, _D), jnp.float32)
  zd = jnp.zeros((_RPW,), jnp.float32)
  xp = jnp.zeros((_NP, _D), jnp.float32).at[:_N].set(x)
  eacat = jnp.concatenate([edge_attr, ea_rev], axis=0)
  dstcat = jnp.concatenate([t_id, h_id])
  srccat = jnp.concatenate([h_id, t_id])

  acc1, deg = _sc_fold(xp, eacat, dstcat, srccat, zr, zd)
  degc = deg.reshape(_NP, 1)
  h1 = _layer_combine(xp, acc1, degc, W_s1, W_n1, r1(b_s1), 1280)

  acc2, _ = _sc_fold(h1, eacat, dstcat, srccat, zr, zd)
  h2 = _layer_combine(h1, acc2, degc, W_s2, W_n2, r1(b_s2), 1280)

  gh1, gh2, gt1, gt2 = _sc_gather4(h1, h2, h_id, t_id)

  logits2d, y2d = _edge_score(
      batch_q_embds, gh1, gh2, edge_attr, gt1, gt2, gn,
      W_p1, r1(b_p1), W_p2, b_p2.reshape(1, 1), 2560)

  ypad = jnp.concatenate(
      [y2d[:, 0], jnp.full((_EPAD - _E,), -jnp.inf, jnp.float32)]
  ).reshape(_EPAD // 128, 128)
  mask2d = _topk_mask(ypad)

  logits = logits2d[:, 0]
  mask = mask2d.reshape(-1)[:_E]
  return (logits, mask)


# speedup vs baseline: 1.2376x; 1.0779x over previous
import functools

import jax
import jax.numpy as jnp
from jax import lax
from jax.experimental import pallas as pl
from jax.experimental.pallas import tpu as pltpu
from jax.experimental.pallas import tpu_sc as plsc

_N = 10000
_E = 320000
_D = 128
_K = 1024
_NP = 10240
_RPW = _NP // 32
_SCH = 2048
_NCH = _E // _SCH
_GCH = 128
_NGC = _E // _GCH
_NW = 32
_SEL = 512


def _lane0():
  return lax.iota(jnp.int32, 16) == 0


def _prefix16(v):
  lanes = lax.iota(jnp.int32, 16)
  dnums = lax.GatherDimensionNumbers(
      offset_dims=(), collapsed_slice_dims=(0,), start_index_map=(0,))
  for s in (1, 2, 4, 8):
    idx = jnp.maximum(lanes - s, 0)
    g = lax.gather(v, idx[:, None], dnums, (1,),
                   mode=lax.GatherScatterMode.PROMISE_IN_BOUNDS)
    v = v + jnp.where(lanes >= s, g, 0)
  return v


def _worker_id():
  ci = lax.axis_index("c")
  si = lax.axis_index("s")
  return si * 2 + ci, ci, si


def _sc_fold_body(h_hbm, eac_hbm, dstc_hbm, srcc_hbm, zr_hbm, zd_hbm,
                  acc_out, deg_out,
                  dstb_v, srcb_v, seld_v, sels_v, selp_v, cntv_v,
                  rowa_v, rowb_v, accl_v, degl_v, sem1, sem2, semd, sems):
  w, ci, si = _worker_id()
  lo = w * _RPW
  pltpu.sync_copy(zr_hbm, accl_v)
  pltpu.sync_copy(zd_hbm, degl_v)
  z16 = jnp.zeros((16,), jnp.int32)

  def zinit(k, carry):
    sels_v[pl.ds(k * 16, 16)] = z16
    selp_v[pl.ds(k * 16, 16)] = z16
    return carry

  lax.fori_loop(0, _SEL // 16, zinit, 0)

  def fold_first(nfold):
    cpa = pltpu.async_copy(h_hbm.at[sels_v.at[pl.ds(0, 128)]], rowa_v, sem1)
    cpb = pltpu.async_copy(eac_hbm.at[selp_v.at[pl.ds(0, 128)]], rowb_v, sem2)
    cpa.wait()
    cpb.wait()

    def fold_one(i, carry):
      @pl.when(i < nfold)
      def _():
        dloc = seld_v[pl.ds(i, 16)][0]
        plsc.addupdate_scatter(degl_v, [jnp.full((16,), dloc, jnp.int32)],
                               jnp.ones((16,), jnp.float32), mask=_lane0())
        for jj in range(8):
          sl = pl.ds(jj * 16, 16)
          plsc.addupdate(accl_v.at[dloc, sl], rowa_v[i, sl] + rowb_v[i, sl])
      return carry

    lax.fori_loop(0, 128, fold_one, 0)

  def shift_sel(k, carry):
    sl_from = pl.ds(128 + k * 16, 16)
    sl_to = pl.ds(k * 16, 16)
    seld_v[sl_to] = seld_v[sl_from]
    sels_v[sl_to] = sels_v[sl_from]
    selp_v[sl_to] = selp_v[sl_from]
    return carry

  def start_load(c, b):
    base = c * _SCH
    pltpu.async_copy(dstc_hbm.at[pl.ds(base, _SCH)], dstb_v.at[b], semd)
    pltpu.async_copy(srcc_hbm.at[pl.ds(base, _SCH)], srcb_v.at[b], sems)

  def wait_load(b):
    pltpu.make_async_copy(dstc_hbm.at[pl.ds(0, _SCH)], dstb_v.at[b],
                          semd).wait()
    pltpu.make_async_copy(srcc_hbm.at[pl.ds(0, _SCH)], srcb_v.at[b],
                          sems).wait()

  def chunk_body(c, cnt):
    base = c * _SCH
    b = lax.rem(c, 2)
    wait_load(b)

    @pl.when(c + 1 < 2 * _NCH)
    def _():
      start_load(c + 1, 1 - b)

    def subvec(j, cnt):
      sl = pl.ds(j * 16, 16)
      d = dstb_v[b, sl]
      m = (d >= lo) & (d < lo + _RPW)
      cadd = plsc.all_reduce_population_count(m)[0]

      @pl.when(cadd > 0)
      def _():
        pos = lax.iota(jnp.int32, 16) + (base + j * 16)
        csum = _prefix16(m.astype(jnp.int32))
        slot = cnt + csum - 1
        plsc.store_scatter(seld_v, [slot], d - lo, mask=m)
        plsc.store_scatter(sels_v, [slot], srcb_v[b, sl], mask=m)
        plsc.store_scatter(selp_v, [slot], pos, mask=m)

      cnt = cnt + cadd
      isfull = cnt >= 128

      @pl.when(isfull)
      def _():
        fold_first(jnp.int32(128))
        shifter = lax.fori_loop(0, 8, shift_sel, 0)

      return jnp.where(isfull, cnt - 128, cnt)

    return lax.fori_loop(0, _SCH // 16, subvec, cnt)

  start_load(0, 0)
  cnt = lax.fori_loop(0, 2 * _NCH, chunk_body, jnp.int32(0))
  fold_first(cnt)

  pltpu.sync_copy(accl_v, acc_out.at[pl.ds(lo, _RPW), :])
  pltpu.sync_copy(degl_v, deg_out.at[pl.ds(lo, _RPW)])


def _sc_gather4_body(h1_hbm, h2_hbm, hid_hbm, tid_hbm,
                     gh1_out, gh2_out, gt1_out, gt2_out,
                     idxh_v, idxt_v, b1_v, b2_v, b3_v, b4_v,
                     sem1, sem2, sem3, sem4):
  w, ci, si = _worker_id()

  def body(j, carry):
    c = w + _NW * j
    base = c * _GCH
    pltpu.sync_copy(hid_hbm.at[pl.ds(base, _GCH)], idxh_v)
    pltpu.sync_copy(tid_hbm.at[pl.ds(base, _GCH)], idxt_v)
    cp1 = pltpu.async_copy(h1_hbm.at[idxh_v], b1_v, sem1)
    cp2 = pltpu.async_copy(h2_hbm.at[idxh_v], b2_v, sem2)
    cp3 = pltpu.async_copy(h1_hbm.at[idxt_v], b3_v, sem3)
    cp4 = pltpu.async_copy(h2_hbm.at[idxt_v], b4_v, sem4)
    cp1.wait()
    pltpu.sync_copy(b1_v, gh1_out.at[pl.ds(base, _GCH), :])
    cp2.wait()
    pltpu.sync_copy(b2_v, gh2_out.at[pl.ds(base, _GCH), :])
    cp3.wait()
    pltpu.sync_copy(b3_v, gt1_out.at[pl.ds(base, _GCH), :])
    cp4.wait()
    pltpu.sync_copy(b4_v, gt2_out.at[pl.ds(base, _GCH), :])
    return carry

  nc = 78 + jnp.where(w < 4, 1, 0).astype(jnp.int32)
  lax.fori_loop(0, nc, body, 0)


@functools.cache
def _sc_kernels():
  mesh = plsc.VectorSubcoreMesh(core_axis_name="c", subcore_axis_name="s",
                                num_cores=2, num_subcores=16)
  params = pltpu.CompilerParams(needs_layout_passes=False)
  fold = functools.partial(
      pl.kernel,
      mesh=mesh,
      compiler_params=params,
      out_type=(
          jax.ShapeDtypeStruct((_NP, _D), jnp.float32),
          jax.ShapeDtypeStruct((_NP,), jnp.float32),
      ),
      scratch_types=[
          pltpu.VMEM((2, _SCH), jnp.int32),
          pltpu.VMEM((2, _SCH), jnp.int32),
          pltpu.VMEM((_SEL,), jnp.int32),
          pltpu.VMEM((_SEL,), jnp.int32),
          pltpu.VMEM((_SEL,), jnp.int32),
          pltpu.VMEM((16,), jnp.int32),
          pltpu.VMEM((128, _D), jnp.float32),
          pltpu.VMEM((128, _D), jnp.float32),
          pltpu.VMEM((_RPW, _D), jnp.float32),
          pltpu.VMEM((_RPW,), jnp.float32),
          pltpu.SemaphoreType.DMA,
          pltpu.SemaphoreType.DMA,
          pltpu.SemaphoreType.DMA,
          pltpu.SemaphoreType.DMA,
      ],
  )(_sc_fold_body)
  gather4 = functools.partial(
      pl.kernel,
      mesh=mesh,
      compiler_params=params,
      out_type=tuple(
          jax.ShapeDtypeStruct((_E, _D), jnp.float32) for _ in range(4)),
      scratch_types=[
          pltpu.VMEM((_GCH,), jnp.int32),
          pltpu.VMEM((_GCH,), jnp.int32),
          pltpu.VMEM((_GCH, _D), jnp.float32),
          pltpu.VMEM((_GCH, _D), jnp.float32),
          pltpu.VMEM((_GCH, _D), jnp.float32),
          pltpu.VMEM((_GCH, _D), jnp.float32),
          pltpu.SemaphoreType.DMA,
          pltpu.SemaphoreType.DMA,
          pltpu.SemaphoreType.DMA,
          pltpu.SemaphoreType.DMA,
      ],
  )(_sc_gather4_body)
  return fold, gather4


def _mlp2_body(x_ref, w1_ref, b1_ref, w2_ref, b2_ref, o_ref):
  h = jnp.maximum(
      jnp.dot(x_ref[...], w1_ref[...], preferred_element_type=jnp.float32)
      + b1_ref[...], 0.0)
  o_ref[...] = (
      jnp.dot(h, w2_ref[...], preferred_element_type=jnp.float32)
      + b2_ref[...])


def _edge_mlp(ea, w1, b1, w2, b2, rb):
  n = ea.shape[0]
  full = lambda i: (0, 0)
  return pl.pallas_call(
      _mlp2_body,
      grid=(n // rb,),
      in_specs=[
          pl.BlockSpec((rb, _D), lambda i: (i, 0)),
          pl.BlockSpec((_D, _D), full),
          pl.BlockSpec((1, _D), full),
          pl.BlockSpec((_D, _D), full),
          pl.BlockSpec((1, _D), full),
      ],
      out_specs=pl.BlockSpec((rb, _D), lambda i: (i, 0)),
      out_shape=jax.ShapeDtypeStruct((n, _D), jnp.float32),
  )(ea, w1, b1, w2, b2)


def _layer_body(h_ref, a_ref, d_ref, ws, wn, b, o_ref):
  deg = jnp.maximum(d_ref[...], 1.0)
  agg = a_ref[...] / deg
  o_ref[...] = jnp.maximum(
      jnp.dot(h_ref[...], ws[...], preferred_element_type=jnp.float32)
      + jnp.dot(agg, wn[...], preferred_element_type=jnp.float32)
      + b[...], 0.0)


def _layer_combine(h, acc, deg, ws, wn, b, rb):
  full = lambda i: (0, 0)
  blk = pl.BlockSpec((rb, _D), lambda i: (i, 0))
  return pl.pallas_call(
      _layer_body,
      grid=(_NP // rb,),
      in_specs=[blk, blk, pl.BlockSpec((rb, 1), lambda i: (i, 0)),
                pl.BlockSpec((_D, _D), full),
                pl.BlockSpec((_D, _D), full),
                pl.BlockSpec((1, _D), full)],
      out_specs=blk,
      out_shape=jax.ShapeDtypeStruct((_NP, _D), jnp.float32),
  )(h, acc, deg, ws, wn, b)


def _score_body(bq_ref, gh1_ref, gh2_ref, ea_ref, gt1_ref, gt2_ref, gn_ref,
                wp1, bp1, wp2, bp2, logit_ref, y_ref):
  ht = jnp.concatenate(
      [bq_ref[...], gh1_ref[...], gh2_ref[...], ea_ref[...],
       gt1_ref[...], gt2_ref[...]], axis=1)
  hid = jnp.maximum(
      jnp.dot(ht, wp1[...], preferred_element_type=jnp.float32)
      + bp1[...], 0.0)
  lg = jnp.dot(hid, wp2[...], preferred_element_type=jnp.float32) + bp2[...]
  logit_ref[...] = lg
  y_ref[...] = lg + gn_ref[...]


def _edge_score(bq, gh1, gh2, ea, gt1, gt2, gn, wp1, bp1, wp2, bp2, rb):
  full = lambda i: (0, 0)
  blk = pl.BlockSpec((rb, _D), lambda i: (i, 0))
  col = pl.BlockSpec((rb, 1), lambda i: (i, 0))
  out = jax.ShapeDtypeStruct((_E, 1), jnp.float32)
  return pl.pallas_call(
      _score_body,
      grid=(_E // rb,),
      in_specs=[blk, blk, blk, blk, blk, blk, col,
                pl.BlockSpec((6 * _D, _D), full),
                pl.BlockSpec((1, _D), full),
                pl.BlockSpec((_D, 1), full),
                pl.BlockSpec((1, 1), full)],
      out_specs=(col, col),
      out_shape=(out, out),
  )(bq, gh1, gh2, ea, gt1, gt2, gn, wp1, bp1, wp2, bp2)


_EPAD = 2560 * 128


def _topk_body(y_ref, o_ref):
  zi = lax.bitcast_convert_type(y_ref[...], jnp.int32)
  z = zi ^ ((zi >> 31) & jnp.int32(0x7FFFFFFF))

  def bis1(i, c):
    lo, hi = c
    half = (lo >> 1) + (hi >> 1)
    mid0 = half + (lo & hi & 1) + ((lo ^ hi) & 1)
    mid = jnp.where(lo < hi, mid0, lo)
    cnt = jnp.sum((z >= mid).astype(jnp.int32))
    pred = cnt >= _K
    return (jnp.where(pred, mid, lo), jnp.where(pred, hi, mid - 1))

  lo, _ = lax.fori_loop(0, 33, bis1, (jnp.int32(-(2 ** 31)),
                                      jnp.int32(2 ** 31 - 1)))
  t = lo
  c_gt = jnp.sum((z > t).astype(jnp.int32))
  r = _K - c_gt
  eq = z == t
  rows = lax.broadcasted_iota(jnp.int32, z.shape, 0)
  cols = lax.broadcasted_iota(jnp.int32, z.shape, 1)
  flat = rows * 128 + cols

  def bis2(i, c):
    lo, hi = c
    mid = jnp.where(lo < hi, (lo + hi) >> 1, lo)
    cnt = jnp.sum((eq & (flat <= mid)).astype(jnp.int32))
    ok = cnt >= r
    return (jnp.where(ok, lo, mid + 1), jnp.where(ok, mid, hi))

  m, _ = lax.fori_loop(0, 20, bis2, (jnp.int32(0), jnp.int32(_EPAD - 1)))
  o_ref[...] = ((z > t) | (eq & (flat <= m))).astype(jnp.float32)


def _topk_mask(ypad):
  return pl.pallas_call(
      _topk_body,
      out_shape=jax.ShapeDtypeStruct((_EPAD // 128, 128), jnp.float32),
  )(ypad)


def kernel(x, edge_index, edge_attr, batch_q_embds, W_pr1, b_pr1, W_pr2,
           b_pr2, W_s1, b_s1, W_n1, W_s2, b_s2, W_n2, W_p1, b_p1, W_p2, b_p2):
  h_id = edge_index[0]
  t_id = edge_index[1]
  r1 = lambda v: v.reshape(1, -1)

  u = jax.random.uniform(jax.random.key(42), (_E,), jnp.float32,
                         1e-10, 1.0 - 1e-10)
  gn = (-jnp.log(-jnp.log(u))).reshape(_E, 1)

  _sc_fold, _sc_gather4 = _sc_kernels()

  ea_rev = _edge_mlp(edge_attr, W_pr1, r1(b_pr1), W_pr2, r1(b_pr2), 2560)

  zr = jnp.zeros((_RPW, _D), jnp.float32)
  zd = jnp.zeros((_RPW,), jnp.float32)
  xp = jnp.zeros((_NP, _D), jnp.float32).at[:_N].set(x)
  eacat = jnp.concatenate([edge_attr, ea_rev], axis=0)
  dstcat = jnp.concatenate([t_id, h_id])
  srccat = jnp.concatenate([h_id, t_id])

  acc1, deg = _sc_fold(xp, eacat, dstcat, srccat, zr, zd)
  degc = deg.reshape(_NP, 1)
  h1 = _layer_combine(xp, acc1, degc, W_s1, W_n1, r1(b_s1), 1280)

  acc2, _ = _sc_fold(h1, eacat, dstcat, srccat, zr, zd)
  h2 = _layer_combine(h1, acc2, degc, W_s2, W_n2, r1(b_s2), 1280)

  gh1, gh2, gt1, gt2 = _sc_gather4(h1, h2, h_id, t_id)

  logits2d, y2d = _edge_score(
      batch_q_embds, gh1, gh2, edge_attr, gt1, gt2, gn,
      W_p1, r1(b_p1), W_p2, b_p2.reshape(1, 1), 2560)

  ypad = jnp.concatenate(
      [y2d[:, 0], jnp.full((_EPAD - _E,), -jnp.inf, jnp.float32)]
  ).reshape(_EPAD // 128, 128)
  mask2d = _topk_mask(ypad)

  logits = logits2d[:, 0]
  mask = mask2d.reshape(-1)[:_E]
  return (logits, mask)
